# baseline (speedup 1.0000x reference)
import jax
import jax.numpy as jnp
from jax.experimental import pallas as pl
from jax.experimental.pallas import tpu as pltpu

B = 64
S = 4
V = 100000
VB = 10240
NBLK = (V + VB - 1) // VB
EPS = 1e-9


def _tc_body(l_ref, u_ref, out_ref, rm_ref, ra_ref):
    j = pl.program_id(0)

    @pl.when(j == 0)
    def _init():
        rm_ref[...] = jnp.full((B, 128), -jnp.inf, jnp.float32)
        ra_ref[...] = jnp.zeros((B, 128), jnp.int32)

    l = l_ref[:, S - 1, :]
    u = u_ref[...]
    g = -jnp.log(-jnp.log(u + EPS) + EPS)
    val = l + g
    col = j * VB + jax.lax.broadcasted_iota(jnp.int32, (B, VB), 1)
    val = jnp.where(col < V, val, -jnp.inf)
    bm = jnp.max(val, axis=1, keepdims=True)
    cand = jnp.where(val == bm, col, jnp.int32(2**31 - 1))
    ba = jnp.min(cand, axis=1, keepdims=True)
    rm = rm_ref[...]
    upd = bm > rm
    ra_ref[...] = jnp.where(upd, ba, ra_ref[...])
    rm_ref[...] = jnp.where(upd, bm, rm)

    @pl.when(j == NBLK - 1)
    def _fin():
        out_ref[...] = ra_ref[:, 0]


def kernel(logits, u):
    out = pl.pallas_call(
        _tc_body,
        grid=(NBLK,),
        in_specs=[
            pl.BlockSpec((B, S, VB), lambda j: (0, 0, j)),
            pl.BlockSpec((B, VB), lambda j: (0, j)),
        ],
        out_specs=pl.BlockSpec((B,), lambda j: (0,)),
        out_shape=jax.ShapeDtypeStruct((B,), jnp.int32),
        scratch_shapes=[
            pltpu.VMEM((B, 128), jnp.float32),
            pltpu.VMEM((B, 128), jnp.int32),
        ],
    )(logits, u)
    return out


# speedup vs baseline: 1.0026x; 1.0026x over previous
import jax
import jax.numpy as jnp
from jax.experimental import pallas as pl
from jax.experimental.pallas import tpu as pltpu

B = 64
S = 4
V = 100000
VB = 11264
NBLK = (V + VB - 1) // VB
EPS = 1e-9


def _tc_body(l_ref, u_ref, out_ref, rm_ref, ra_ref):
    j = pl.program_id(0)

    @pl.when(j == 0)
    def _init():
        rm_ref[...] = jnp.full((B, 128), -jnp.inf, jnp.float32)
        ra_ref[...] = jnp.zeros((B, 128), jnp.int32)

    l = l_ref[:, S - 1, :]
    u = u_ref[...]
    g = -jnp.log(-jnp.log(u + EPS) + EPS)
    val = l + g
    col = j * VB + jax.lax.broadcasted_iota(jnp.int32, (B, VB), 1)
    val = jnp.where(col < V, val, -jnp.inf)
    bm = jnp.max(val, axis=1, keepdims=True)
    cand = jnp.where(val == bm, col, jnp.int32(2**31 - 1))
    ba = jnp.min(cand, axis=1, keepdims=True)
    rm = rm_ref[...]
    upd = bm > rm
    ra_ref[...] = jnp.where(upd, ba, ra_ref[...])
    rm_ref[...] = jnp.where(upd, bm, rm)

    @pl.when(j == NBLK - 1)
    def _fin():
        out_ref[...] = ra_ref[:, 0]


def kernel(logits, u):
    out = pl.pallas_call(
        _tc_body,
        grid=(NBLK,),
        in_specs=[
            pl.BlockSpec((B, S, VB), lambda j: (0, 0, j)),
            pl.BlockSpec((B, VB), lambda j: (0, j)),
        ],
        out_specs=pl.BlockSpec((B,), lambda j: (0,)),
        out_shape=jax.ShapeDtypeStruct((B,), jnp.int32),
        scratch_shapes=[
            pltpu.VMEM((B, 128), jnp.float32),
            pltpu.VMEM((B, 128), jnp.int32),
        ],
    )(logits, u)
    return out
